# baseline (speedup 1.0000x reference)
import functools

import jax
import jax.numpy as jnp
from jax import lax
from jax.experimental import pallas as pl
from jax.experimental.pallas import tpu as pltpu
from jax.experimental.pallas import tpu_sc as plsc

PAD_ID = 0
IGNORE_ID = -100
LANES = 16
NBUF = 2


def _sc_embed(dec_flat, wte, wpe):
    N = dec_flat.shape[0]
    D = wte.shape[1]
    T = wpe.shape[0]
    info = plsc.get_sparse_core_info()
    nw = info.num_cores * info.num_subcores
    per_w = N // nw
    C = 32
    n_chunks = per_w // C
    mesh = plsc.VectorSubcoreMesh(core_axis_name="c", subcore_axis_name="s")

    @functools.partial(
        pl.kernel,
        mesh=mesh,
        out_type=(
            jax.ShapeDtypeStruct((N, D), jnp.float32),
            jax.ShapeDtypeStruct((N,), jnp.int32),
            jax.ShapeDtypeStruct((N,), jnp.int32),
        ),
        scratch_types=[
            pltpu.VMEM((per_w,), jnp.int32),
            pltpu.VMEM((per_w,), jnp.int32),
            pltpu.VMEM((per_w,), jnp.int32),
            pltpu.VMEM((NBUF, C, D), jnp.float32),
            pltpu.VMEM((NBUF, C, D), jnp.float32),
            pltpu.SemaphoreType.DMA,
            pltpu.SemaphoreType.DMA,
            pltpu.SemaphoreType.DMA,
            pltpu.SemaphoreType.DMA,
            pltpu.SemaphoreType.DMA,
            pltpu.SemaphoreType.DMA,
        ],
    )
    def k(dec_hbm, wte_hbm, wpe_hbm, tok_hbm, din_hbm, keep_hbm,
          dec_v, din_v, keep_v, rows_v, wpe_v,
          sg0, sg1, sw0, sw1, so0, so1):
        gsems = (sg0, sg1)
        wsems = (sw0, sw1)
        osems = (so0, so1)
        wid = lax.axis_index("s") * info.num_cores + lax.axis_index("c")
        base = wid * per_w
        t0 = lax.rem(base, T)

        pltpu.sync_copy(dec_hbm.at[pl.ds(base, per_w)], dec_v)
        ign = jnp.full((LANES,), IGNORE_ID, jnp.int32)
        pad = jnp.full((LANES,), PAD_ID, jnp.int32)
        one = jnp.full((LANES,), 1, jnp.int32)
        for i in range(per_w // LANES):
            sl = pl.ds(i * LANES, LANES)
            v = dec_v[sl]
            m = v != ign
            din_v[sl] = jnp.where(m, v, pad)
            keep_v[sl] = jnp.where(m, one, pad)
        dout = pltpu.async_copy(din_v, din_hbm.at[pl.ds(base, per_w)], so0)
        kout = pltpu.async_copy(keep_v, keep_hbm.at[pl.ds(base, per_w)], so1)

        gather_cp = [None] * NBUF
        wpe_cp = [None] * NBUF
        out_cp = [None] * NBUF

        def start_chunk(g):
            b = g % NBUF
            gather_cp[b] = pltpu.async_copy(
                wte_hbm.at[din_v.at[pl.ds(g * C, C)]], rows_v.at[b], gsems[b])
            tb = pl.multiple_of(t0 + g * C, C)
            wpe_cp[b] = pltpu.async_copy(
                wpe_hbm.at[pl.ds(tb, C)], wpe_v.at[b], wsems[b])

        def finish_chunk(g):
            b = g % NBUF
            gather_cp[b].wait()
            wpe_cp[b].wait()

            def add_row(r, cc):
                for j in range(D // LANES):
                    sl = pl.ds(j * LANES, LANES)
                    rows_v[b, r, sl] = rows_v[b, r, sl] + wpe_v[b, r, sl]
                return cc

            lax.fori_loop(0, C, add_row, 0)
            rb = pl.multiple_of(base + g * C, C)
            out_cp[b] = pltpu.async_copy(
                rows_v.at[b], tok_hbm.at[pl.ds(rb, C)], osems[b])

        dout.wait()
        kout.wait()
        start_chunk(0)
        for g in range(n_chunks):
            if g + 1 < n_chunks:
                if g >= 1:
                    out_cp[(g + 1) % NBUF].wait()
                start_chunk(g + 1)
            finish_chunk(g)
        out_cp[(n_chunks - 2) % NBUF].wait()
        out_cp[(n_chunks - 1) % NBUF].wait()

    return k(dec_flat, wte, wpe)


def kernel(enc_hid, dec_or_lab, metadata, wte, wpe):
    B, T = dec_or_lab.shape
    D = wte.shape[1]
    dec_flat = dec_or_lab.reshape(B * T)
    tok, din, keep = _sc_embed(dec_flat, wte, wpe[:T])
    token_emb = tok.reshape(B, T, D)
    keep_b = keep.reshape(B, T).astype(bool)
    dec_in = din.reshape(B, T)
    enc_mask_2d = jnp.ones((B, T), dtype=bool)
    return (enc_hid, token_emb, enc_mask_2d, keep_b, metadata, dec_in, keep_b)


# speedup vs baseline: 1.0411x; 1.0411x over previous
import functools

import jax
import jax.numpy as jnp
from jax import lax
from jax.experimental import pallas as pl
from jax.experimental.pallas import tpu as pltpu
from jax.experimental.pallas import tpu_sc as plsc

PAD_ID = 0
IGNORE_ID = -100
LANES = 16
NBUF = 2


def _sc_embed(dec_flat, wte, wpe, batch):
    N = dec_flat.shape[0]
    D = wte.shape[1]
    T = wpe.shape[0]
    info = plsc.get_sparse_core_info()
    nw = info.num_cores * info.num_subcores
    per_w = N // nw
    tw = T // nw
    C = 32
    hpb = tw // C
    n_chunks = per_w // C
    mesh = plsc.VectorSubcoreMesh(core_axis_name="c", subcore_axis_name="s")

    @functools.partial(
        pl.kernel,
        mesh=mesh,
        out_type=(
            jax.ShapeDtypeStruct((N, D), jnp.float32),
            jax.ShapeDtypeStruct((N,), jnp.int32),
            jax.ShapeDtypeStruct((N,), jnp.int32),
        ),
        scratch_types=[
            pltpu.VMEM((per_w,), jnp.int32),
            pltpu.VMEM((per_w,), jnp.int32),
            pltpu.VMEM((per_w,), jnp.int32),
            pltpu.VMEM((NBUF, C, D), jnp.float32),
            pltpu.VMEM((tw, D), jnp.float32),
            pltpu.SemaphoreType.DMA,
            pltpu.SemaphoreType.DMA,
            pltpu.SemaphoreType.DMA,
            pltpu.SemaphoreType.DMA,
            pltpu.SemaphoreType.DMA,
        ],
    )
    def k(dec_hbm, wte_hbm, wpe_hbm, tok_hbm, din_hbm, keep_hbm,
          dec_v, din_v, keep_v, rows_v, wpe_v,
          sg0, sg1, so0, so1, sw):
        gsems = (sg0, sg1)
        osems = (so0, so1)
        wid = lax.axis_index("s") * info.num_cores + lax.axis_index("c")
        t0 = pl.multiple_of(wid * tw, tw)

        wcp = pltpu.async_copy(wpe_hbm.at[pl.ds(t0, tw)], wpe_v, sw)

        for b in range(batch):
            seg = pl.multiple_of(b * T + t0, tw)
            pltpu.sync_copy(dec_hbm.at[pl.ds(seg, tw)],
                            dec_v.at[pl.ds(b * tw, tw)])
        ign = jnp.full((LANES,), IGNORE_ID, jnp.int32)
        pad = jnp.full((LANES,), PAD_ID, jnp.int32)
        one = jnp.full((LANES,), 1, jnp.int32)
        for i in range(per_w // LANES):
            sl = pl.ds(i * LANES, LANES)
            v = dec_v[sl]
            m = v != ign
            din_v[sl] = jnp.where(m, v, pad)
            keep_v[sl] = jnp.where(m, one, pad)
        small_cp = []
        for b in range(batch):
            seg = pl.multiple_of(b * T + t0, tw)
            lo = pl.ds(b * tw, tw)
            small_cp.append(pltpu.async_copy(din_v.at[lo],
                                             din_hbm.at[pl.ds(seg, tw)], so0))
            small_cp.append(pltpu.async_copy(keep_v.at[lo],
                                             keep_hbm.at[pl.ds(seg, tw)], so1))

        gather_cp = [None] * NBUF
        out_cp = [None] * NBUF

        def start_chunk(g):
            b = g % NBUF
            gather_cp[b] = pltpu.async_copy(
                wte_hbm.at[din_v.at[pl.ds(g * C, C)]], rows_v.at[b], gsems[b])

        def finish_chunk(g):
            b = g % NBUF
            woff = (g % hpb) * C
            gather_cp[b].wait()

            def add_row(r, cc):
                for j in range(D // LANES):
                    sl = pl.ds(j * LANES, LANES)
                    rows_v[b, r, sl] = rows_v[b, r, sl] + wpe_v[woff + r, sl]
                return cc

            lax.fori_loop(0, C, add_row, 0)
            rb = pl.multiple_of((g // hpb) * T + t0 + woff, C)
            out_cp[b] = pltpu.async_copy(
                rows_v.at[b], tok_hbm.at[pl.ds(rb, C)], osems[b])

        for cp in small_cp:
            cp.wait()
        wcp.wait()
        start_chunk(0)
        for g in range(n_chunks):
            if g + 1 < n_chunks:
                if g >= 1:
                    out_cp[(g + 1) % NBUF].wait()
                start_chunk(g + 1)
            finish_chunk(g)
        out_cp[(n_chunks - 2) % NBUF].wait()
        out_cp[(n_chunks - 1) % NBUF].wait()

    return k(dec_flat, wte, wpe)


def kernel(enc_hid, dec_or_lab, metadata, wte, wpe):
    B, T = dec_or_lab.shape
    D = wte.shape[1]
    dec_flat = dec_or_lab.reshape(B * T)
    tok, din, keep = _sc_embed(dec_flat, wte, wpe[:T], B)
    token_emb = tok.reshape(B, T, D)
    keep_b = keep.reshape(B, T).astype(bool)
    dec_in = din.reshape(B, T)
    enc_mask_2d = jnp.ones((B, T), dtype=bool)
    return (enc_hid, token_emb, enc_mask_2d, keep_b, metadata, dec_in, keep_b)


# speedup vs baseline: 1.1240x; 1.0796x over previous
import functools

import jax
import jax.numpy as jnp
from jax import lax
from jax.experimental import pallas as pl
from jax.experimental.pallas import tpu as pltpu
from jax.experimental.pallas import tpu_sc as plsc

PAD_ID = 0
IGNORE_ID = -100
LANES = 16
NBUF = 2


def _sc_embed(dec_flat, wte, wpe, batch):
    N = dec_flat.shape[0]
    D = wte.shape[1]
    T = wpe.shape[0]
    info = plsc.get_sparse_core_info()
    nw = info.num_cores * info.num_subcores
    per_w = N // nw
    tw = T // nw
    C = 32
    hpb = tw // C
    n_chunks = per_w // C
    mesh = plsc.VectorSubcoreMesh(core_axis_name="c", subcore_axis_name="s")

    @functools.partial(
        pl.kernel,
        mesh=mesh,
        out_type=(
            jax.ShapeDtypeStruct((N, D), jnp.float32),
            jax.ShapeDtypeStruct((N,), jnp.int32),
            jax.ShapeDtypeStruct((N,), jnp.int32),
        ),
        scratch_types=[
            pltpu.VMEM((per_w,), jnp.int32),
        ] + [pltpu.VMEM((C,), jnp.int32) for _ in range(n_chunks)] + [
            pltpu.VMEM((n_chunks, C), jnp.int32),
            pltpu.VMEM((NBUF, C, D), jnp.float32),
            pltpu.VMEM((tw, D), jnp.float32),
            pltpu.SemaphoreType.DMA,
            pltpu.SemaphoreType.DMA,
            pltpu.SemaphoreType.DMA,
            pltpu.SemaphoreType.DMA,
            pltpu.SemaphoreType.DMA,
        ],
    )
    def k(dec_hbm, wte_hbm, wpe_hbm, tok_hbm, din_hbm, keep_hbm,
          dec_v, *rest):
        din_vs = rest[:n_chunks]
        keep_v, rows_v, wpe_v, sg0, sg1, so0, so1, sw = rest[n_chunks:]
        gsems = (sg0, sg1)
        osems = (so0, so1)
        wid = lax.axis_index("s") * info.num_cores + lax.axis_index("c")
        t0 = pl.multiple_of(wid * tw, tw)

        wcp = pltpu.async_copy(wpe_hbm.at[pl.ds(t0, tw)], wpe_v, sw)

        def hbm_row(g):
            return pl.multiple_of((g // hpb) * T + t0 + (g % hpb) * C, C)

        dec_cp = []
        for b in range(batch):
            seg = pl.multiple_of(b * T + t0, tw)
            dec_cp.append(pltpu.async_copy(dec_hbm.at[pl.ds(seg, tw)],
                                           dec_v.at[pl.ds(b * tw, tw)], so0))
        for cp in dec_cp:
            cp.wait()
        ign = jnp.full((LANES,), IGNORE_ID, jnp.int32)
        pad = jnp.full((LANES,), PAD_ID, jnp.int32)
        one = jnp.full((LANES,), 1, jnp.int32)
        small_cp = []
        for g in range(n_chunks):
            for i in range(C // LANES):
                sl = pl.ds(i * LANES, LANES)
                v = dec_v[pl.ds(g * C + i * LANES, LANES)]
                m = v != ign
                din_vs[g][sl] = jnp.where(m, v, pad)
                keep_v[g, sl] = jnp.where(m, one, pad)
            small_cp.append(pltpu.async_copy(
                din_vs[g], din_hbm.at[pl.ds(hbm_row(g), C)], so0))
            small_cp.append(pltpu.async_copy(
                keep_v.at[g], keep_hbm.at[pl.ds(hbm_row(g), C)], so1))

        gather_cp = [None] * NBUF
        out_cp = [None] * NBUF

        def start_chunk(g):
            b = g % NBUF
            gather_cp[b] = pltpu.async_copy(
                wte_hbm.at[din_vs[g]], rows_v.at[b], gsems[b])

        def finish_chunk(g):
            b = g % NBUF
            woff = (g % hpb) * C
            gather_cp[b].wait()

            def add_row(r, cc):
                for j in range(D // LANES):
                    sl = pl.ds(j * LANES, LANES)
                    plsc.addupdate(rows_v.at[b, r, sl], wpe_v[woff + r, sl])
                return cc

            lax.fori_loop(0, C, add_row, 0)
            out_cp[b] = pltpu.async_copy(
                rows_v.at[b], tok_hbm.at[pl.ds(hbm_row(g), C)], osems[b])

        for cp in small_cp:
            cp.wait()
        wcp.wait()
        start_chunk(0)
        for g in range(n_chunks):
            if g + 1 < n_chunks:
                if g >= 1:
                    out_cp[(g + 1) % NBUF].wait()
                start_chunk(g + 1)
            finish_chunk(g)
        out_cp[(n_chunks - 2) % NBUF].wait()
        out_cp[(n_chunks - 1) % NBUF].wait()

    return k(dec_flat, wte, wpe)


def kernel(enc_hid, dec_or_lab, metadata, wte, wpe):
    B, T = dec_or_lab.shape
    D = wte.shape[1]
    dec_flat = dec_or_lab.reshape(B * T)
    tok, din, keep = _sc_embed(dec_flat, wte, wpe[:T], B)
    token_emb = tok.reshape(B, T, D)
    keep_b = keep.reshape(B, T).astype(bool)
    dec_in = din.reshape(B, T)
    enc_mask_2d = jnp.ones((B, T), dtype=bool)
    return (enc_hid, token_emb, enc_mask_2d, keep_b, metadata, dec_in, keep_b)


# speedup vs baseline: 1.1554x; 1.0279x over previous
import functools

import jax
import jax.numpy as jnp
from jax import lax
from jax.experimental import pallas as pl
from jax.experimental.pallas import tpu as pltpu
from jax.experimental.pallas import tpu_sc as plsc

PAD_ID = 0
IGNORE_ID = -100
LANES = 16
NBUF = 3


def _sc_embed(dec_flat, wte, wpe, batch):
    N = dec_flat.shape[0]
    D = wte.shape[1]
    T = wpe.shape[0]
    info = plsc.get_sparse_core_info()
    nw = info.num_cores * info.num_subcores
    per_w = N // nw
    tw = T // nw
    C = 32
    hpb = tw // C
    n_chunks = per_w // C
    mesh = plsc.VectorSubcoreMesh(core_axis_name="c", subcore_axis_name="s")

    @functools.partial(
        pl.kernel,
        mesh=mesh,
        out_type=(
            jax.ShapeDtypeStruct((N, D), jnp.float32),
            jax.ShapeDtypeStruct((N,), jnp.int32),
            jax.ShapeDtypeStruct((N,), jnp.int32),
        ),
        scratch_types=[
            pltpu.VMEM((per_w,), jnp.int32),
        ] + [pltpu.VMEM((C,), jnp.int32) for _ in range(n_chunks)] + [
            pltpu.VMEM((n_chunks, C), jnp.int32),
            pltpu.VMEM((NBUF, C, D), jnp.float32),
            pltpu.VMEM((tw, D), jnp.float32),
        ] + [pltpu.SemaphoreType.DMA for _ in range(2 * NBUF + 1)],
    )
    def k(dec_hbm, wte_hbm, wpe_hbm, tok_hbm, din_hbm, keep_hbm,
          dec_v, *rest):
        din_vs = rest[:n_chunks]
        keep_v, rows_v, wpe_v = rest[n_chunks:n_chunks + 3]
        sems = rest[n_chunks + 3:]
        gsems = sems[:NBUF]
        osems = sems[NBUF:2 * NBUF]
        sw = sems[2 * NBUF]
        so0, so1 = osems[0], osems[1]
        wid = lax.axis_index("s") * info.num_cores + lax.axis_index("c")
        t0 = pl.multiple_of(wid * tw, tw)

        wcp = pltpu.async_copy(wpe_hbm.at[pl.ds(t0, tw)], wpe_v, sw)

        def hbm_row(g):
            return pl.multiple_of((g // hpb) * T + t0 + (g % hpb) * C, C)

        dec_cp = []
        for b in range(batch):
            seg = pl.multiple_of(b * T + t0, tw)
            dec_cp.append(pltpu.async_copy(dec_hbm.at[pl.ds(seg, tw)],
                                           dec_v.at[pl.ds(b * tw, tw)], so0))
        for cp in dec_cp:
            cp.wait()
        ign = jnp.full((LANES,), IGNORE_ID, jnp.int32)
        pad = jnp.full((LANES,), PAD_ID, jnp.int32)
        one = jnp.full((LANES,), 1, jnp.int32)
        small_cp = []
        for g in range(n_chunks):
            for i in range(C // LANES):
                sl = pl.ds(i * LANES, LANES)
                v = dec_v[pl.ds(g * C + i * LANES, LANES)]
                m = v != ign
                din_vs[g][sl] = jnp.where(m, v, pad)
                keep_v[g, sl] = jnp.where(m, one, pad)
            small_cp.append(pltpu.async_copy(
                din_vs[g], din_hbm.at[pl.ds(hbm_row(g), C)], so0))
            small_cp.append(pltpu.async_copy(
                keep_v.at[g], keep_hbm.at[pl.ds(hbm_row(g), C)], so1))

        gather_cp = [None] * NBUF
        out_cp = [None] * NBUF

        def start_chunk(g):
            b = g % NBUF
            gather_cp[b] = pltpu.async_copy(
                wte_hbm.at[din_vs[g]], rows_v.at[b], gsems[b])

        def finish_chunk(g):
            b = g % NBUF
            woff = (g % hpb) * C
            gather_cp[b].wait()

            def add_row(r, cc):
                for j in range(D // LANES):
                    sl = pl.ds(j * LANES, LANES)
                    plsc.addupdate(rows_v.at[b, r, sl], wpe_v[woff + r, sl])
                return cc

            lax.fori_loop(0, C, add_row, 0)
            out_cp[b] = pltpu.async_copy(
                rows_v.at[b], tok_hbm.at[pl.ds(hbm_row(g), C)], osems[b])

        for cp in small_cp:
            cp.wait()
        wcp.wait()
        for g in range(NBUF - 1):
            start_chunk(g)
        for g in range(n_chunks):
            nxt = g + NBUF - 1
            if nxt < n_chunks:
                if nxt >= NBUF:
                    out_cp[nxt % NBUF].wait()
                start_chunk(nxt)
            finish_chunk(g)
        for g in range(n_chunks - NBUF, n_chunks):
            out_cp[g % NBUF].wait()

    return k(dec_flat, wte, wpe)


def kernel(enc_hid, dec_or_lab, metadata, wte, wpe):
    B, T = dec_or_lab.shape
    D = wte.shape[1]
    dec_flat = dec_or_lab.reshape(B * T)
    tok, din, keep = _sc_embed(dec_flat, wte, wpe[:T], B)
    token_emb = tok.reshape(B, T, D)
    keep_b = keep.reshape(B, T).astype(bool)
    dec_in = din.reshape(B, T)
    enc_mask_2d = jnp.ones((B, T), dtype=bool)
    return (enc_hid, token_emb, enc_mask_2d, keep_b, metadata, dec_in, keep_b)


# speedup vs baseline: 1.1762x; 1.0180x over previous
import functools

import jax
import jax.numpy as jnp
from jax import lax
from jax.experimental import pallas as pl
from jax.experimental.pallas import tpu as pltpu
from jax.experimental.pallas import tpu_sc as plsc

PAD_ID = 0
IGNORE_ID = -100
LANES = 16
NBUF = 3


def _sc_embed(dec_flat, wte, wpe, batch):
    N = dec_flat.shape[0]
    D = wte.shape[1]
    T = wpe.shape[0]
    info = plsc.get_sparse_core_info()
    nw = info.num_cores * info.num_subcores
    per_w = N // nw
    tw = T // nw
    C = 32
    hpb = tw // C
    n_chunks = per_w // C
    mesh = plsc.VectorSubcoreMesh(core_axis_name="c", subcore_axis_name="s")

    @functools.partial(
        pl.kernel,
        mesh=mesh,
        out_type=(
            jax.ShapeDtypeStruct((N, D), jnp.float32),
            jax.ShapeDtypeStruct((N,), jnp.int32),
            jax.ShapeDtypeStruct((N,), jnp.int32),
        ),
        scratch_types=[
            pltpu.VMEM((per_w,), jnp.int32),
        ] + [pltpu.VMEM((C,), jnp.int32) for _ in range(n_chunks)] + [
            pltpu.VMEM((n_chunks, C), jnp.int32),
            pltpu.VMEM((NBUF, C, D), jnp.float32),
            pltpu.VMEM((tw, D), jnp.float32),
        ] + [pltpu.SemaphoreType.DMA for _ in range(2 * NBUF + 3)],
    )
    def k(dec_hbm, wte_hbm, wpe_hbm, tok_hbm, din_hbm, keep_hbm,
          dec_v, *rest):
        din_vs = rest[:n_chunks]
        keep_v, rows_v, wpe_v = rest[n_chunks:n_chunks + 3]
        sems = rest[n_chunks + 3:]
        gsems = sems[:NBUF]
        osems = sems[NBUF:2 * NBUF]
        sw, sd0, sd1 = sems[2 * NBUF:]
        wid = lax.axis_index("s") * info.num_cores + lax.axis_index("c")
        t0 = pl.multiple_of(wid * tw, tw)

        wcp = pltpu.async_copy(wpe_hbm.at[pl.ds(t0, tw)], wpe_v, sw)

        def hbm_row(g):
            return pl.multiple_of((g // hpb) * T + t0 + (g % hpb) * C, C)

        dec_cp = []
        for b in range(batch):
            seg = pl.multiple_of(b * T + t0, tw)
            dec_cp.append(pltpu.async_copy(dec_hbm.at[pl.ds(seg, tw)],
                                           dec_v.at[pl.ds(b * tw, tw)], sd0))
        for cp in dec_cp:
            cp.wait()
        ign = jnp.full((LANES,), IGNORE_ID, jnp.int32)
        pad = jnp.full((LANES,), PAD_ID, jnp.int32)
        one = jnp.full((LANES,), 1, jnp.int32)

        def mask_block(g):
            for i in range(C // LANES):
                sl = pl.ds(i * LANES, LANES)
                v = dec_v[pl.ds(g * C + i * LANES, LANES)]
                m = v != ign
                din_vs[g][sl] = jnp.where(m, v, pad)
                keep_v[g, sl] = jnp.where(m, one, pad)

        gather_cp = [None] * NBUF
        out_cp = [None] * NBUF

        def start_chunk(g):
            b = g % NBUF
            gather_cp[b] = pltpu.async_copy(
                wte_hbm.at[din_vs[g]], rows_v.at[b], gsems[b])

        def finish_chunk(g):
            b = g % NBUF
            woff = (g % hpb) * C
            gather_cp[b].wait()

            def add_row(r, cc):
                for j in range(D // LANES):
                    sl = pl.ds(j * LANES, LANES)
                    plsc.addupdate(rows_v.at[b, r, sl], wpe_v[woff + r, sl])
                return cc

            lax.fori_loop(0, C, add_row, 0)
            out_cp[b] = pltpu.async_copy(
                rows_v.at[b], tok_hbm.at[pl.ds(hbm_row(g), C)], osems[b])

        for g in range(NBUF - 1):
            mask_block(g)
            start_chunk(g)
        for g in range(NBUF - 1, n_chunks):
            mask_block(g)
        small_cp = []
        for g in range(n_chunks):
            small_cp.append(pltpu.async_copy(
                din_vs[g], din_hbm.at[pl.ds(hbm_row(g), C)], sd0))
            small_cp.append(pltpu.async_copy(
                keep_v.at[g], keep_hbm.at[pl.ds(hbm_row(g), C)], sd1))
        wcp.wait()
        for g in range(n_chunks):
            nxt = g + NBUF - 1
            if nxt < n_chunks:
                if nxt >= NBUF:
                    out_cp[nxt % NBUF].wait()
                start_chunk(nxt)
            finish_chunk(g)
        for g in range(n_chunks - NBUF, n_chunks):
            out_cp[g % NBUF].wait()
        for cp in small_cp:
            cp.wait()

    return k(dec_flat, wte, wpe)


def kernel(enc_hid, dec_or_lab, metadata, wte, wpe):
    B, T = dec_or_lab.shape
    D = wte.shape[1]
    dec_flat = dec_or_lab.reshape(B * T)
    tok, din, keep = _sc_embed(dec_flat, wte, wpe[:T], B)
    token_emb = tok.reshape(B, T, D)
    keep_b = keep.reshape(B, T).astype(bool)
    dec_in = din.reshape(B, T)
    enc_mask_2d = jnp.ones((B, T), dtype=bool)
    return (enc_hid, token_emb, enc_mask_2d, keep_b, metadata, dec_in, keep_b)


# speedup vs baseline: 1.2763x; 1.0851x over previous
import functools

import jax
import jax.numpy as jnp
from jax import lax
from jax.experimental import pallas as pl
from jax.experimental.pallas import tpu as pltpu
from jax.experimental.pallas import tpu_sc as plsc

PAD_ID = 0
IGNORE_ID = -100
LANES = 16
NBUF = 3


def _sc_embed(dec_flat, wte, wpe, batch):
    N = dec_flat.shape[0]
    D = wte.shape[1]
    T = wpe.shape[0]
    info = plsc.get_sparse_core_info()
    nw = info.num_cores * info.num_subcores
    per_w = N // nw
    tw = T // nw
    C = 32
    hpb = tw // C
    n_chunks = per_w // C
    mesh = plsc.VectorSubcoreMesh(core_axis_name="c", subcore_axis_name="s")

    @functools.partial(
        pl.kernel,
        mesh=mesh,
        out_type=(
            jax.ShapeDtypeStruct((N, D), jnp.float32),
            jax.ShapeDtypeStruct((N,), jnp.int32),
            jax.ShapeDtypeStruct((N,), jnp.int32),
        ),
        scratch_types=[
            pltpu.VMEM((per_w,), jnp.int32),
        ] + [pltpu.VMEM((C,), jnp.int32) for _ in range(n_chunks)] + [
            pltpu.VMEM((n_chunks, C), jnp.int32),
            pltpu.VMEM((NBUF, C, D), jnp.float32),
            pltpu.VMEM((tw, D), jnp.float32),
        ] + [pltpu.SemaphoreType.DMA for _ in range(2 * NBUF + 3)],
    )
    def k(dec_hbm, wte_hbm, wpe_hbm, tok_hbm, din_hbm, keep_hbm,
          dec_v, *rest):
        din_vs = rest[:n_chunks]
        keep_v, rows_v, wpe_v = rest[n_chunks:n_chunks + 3]
        sems = rest[n_chunks + 3:]
        gsems = sems[:NBUF]
        osems = sems[NBUF:2 * NBUF]
        sw, sd0, sd1 = sems[2 * NBUF:]
        wid = lax.axis_index("s") * info.num_cores + lax.axis_index("c")
        t0 = pl.multiple_of(wid * tw, tw)

        wcp = pltpu.async_copy(wpe_hbm.at[pl.ds(t0, tw)], wpe_v, sw)

        def hbm_row(g):
            return pl.multiple_of((g // hpb) * T + t0 + (g % hpb) * C, C)

        dec_cp = []
        for b in range(batch):
            seg = pl.multiple_of(b * T + t0, tw)
            dec_cp.append(pltpu.async_copy(dec_hbm.at[pl.ds(seg, tw)],
                                           dec_v.at[pl.ds(b * tw, tw)], sd0))
        for cp in dec_cp:
            cp.wait()
        ign = jnp.full((LANES,), IGNORE_ID, jnp.int32)
        pad = jnp.full((LANES,), PAD_ID, jnp.int32)
        one = jnp.full((LANES,), 1, jnp.int32)

        def mask_block(g):
            for i in range(C // LANES):
                sl = pl.ds(i * LANES, LANES)
                v = dec_v[pl.ds(g * C + i * LANES, LANES)]
                m = v != ign
                din_vs[g][sl] = jnp.where(m, v, pad)
                keep_v[g, sl] = jnp.where(m, one, pad)

        gather_cp = [None] * NBUF
        out_cp = [None] * NBUF

        def start_chunk(g):
            b = g % NBUF
            gather_cp[b] = pltpu.async_copy(
                wte_hbm.at[din_vs[g]], rows_v.at[b], gsems[b])

        def finish_chunk(g):
            b = g % NBUF
            woff = (g % hpb) * C
            gather_cp[b].wait()

            @plsc.parallel_loop(0, C, step=1, unroll=2)
            def add_row(r):
                for j in range(D // LANES):
                    sl = pl.ds(j * LANES, LANES)
                    plsc.addupdate(rows_v.at[b, r, sl], wpe_v[woff + r, sl])
            out_cp[b] = pltpu.async_copy(
                rows_v.at[b], tok_hbm.at[pl.ds(hbm_row(g), C)], osems[b])

        for g in range(NBUF - 1):
            mask_block(g)
            start_chunk(g)
        for g in range(NBUF - 1, n_chunks):
            mask_block(g)
        small_cp = []
        for g in range(n_chunks):
            small_cp.append(pltpu.async_copy(
                din_vs[g], din_hbm.at[pl.ds(hbm_row(g), C)], sd0))
            small_cp.append(pltpu.async_copy(
                keep_v.at[g], keep_hbm.at[pl.ds(hbm_row(g), C)], sd1))
        wcp.wait()
        for g in range(n_chunks):
            nxt = g + NBUF - 1
            if nxt < n_chunks:
                if nxt >= NBUF:
                    out_cp[nxt % NBUF].wait()
                start_chunk(nxt)
            finish_chunk(g)
        for g in range(n_chunks - NBUF, n_chunks):
            out_cp[g % NBUF].wait()
        for cp in small_cp:
            cp.wait()

    return k(dec_flat, wte, wpe)


def kernel(enc_hid, dec_or_lab, metadata, wte, wpe):
    B, T = dec_or_lab.shape
    D = wte.shape[1]
    dec_flat = dec_or_lab.reshape(B * T)
    tok, din, keep = _sc_embed(dec_flat, wte, wpe[:T], B)
    token_emb = tok.reshape(B, T, D)
    keep_b = keep.reshape(B, T).astype(bool)
    dec_in = din.reshape(B, T)
    enc_mask_2d = jnp.ones((B, T), dtype=bool)
    return (enc_hid, token_emb, enc_mask_2d, keep_b, metadata, dec_in, keep_b)
